# baseline (speedup 1.0000x reference)
import functools

import jax
import jax.numpy as jnp
from jax.experimental import pallas as pl
from jax.experimental.pallas import tpu as pltpu

_BEAM = 128
_RB = 16
_CB = 8192
_DEPTH = _CB // 128
_NEG = float("-inf")
_IMAX = jnp.iinfo(jnp.int32).max


def _gather_body(idx_ref, row_ref, out_ref):
    out_ref[...] = row_ref[...]


def _before(v0, i0, v1, i1):
    return (v0 > v1) | ((v0 == v1) & (i0 < i1))


def _stage(v, ix, j, k, asc):
    rb, n, lanes = v.shape
    m = n // (2 * j)
    v4 = v.reshape(rb, m, 2, j, lanes)
    i4 = ix.reshape(rb, m, 2, j, lanes)
    v0, v1 = v4[:, :, 0], v4[:, :, 1]
    i0, i1 = i4[:, :, 0], i4[:, :, 1]
    b = _before(v0, i0, v1, i1)
    a = jax.lax.broadcasted_iota(jnp.int32, (rb, m, j, lanes), 1)
    dirbit = ((a * (2 * j)) & k) == 0
    sel = (b != dirbit) if asc else (b == dirbit)
    fv = jnp.where(sel, v0, v1)
    sv = jnp.where(sel, v1, v0)
    fi = jnp.where(sel, i0, i1)
    si = jnp.where(sel, i1, i0)
    v = jnp.concatenate([fv[:, :, None], sv[:, :, None]], axis=2)
    ix = jnp.concatenate([fi[:, :, None], si[:, :, None]], axis=2)
    return v.reshape(rb, n, lanes), ix.reshape(rb, n, lanes)


def _rev(x):
    rb, n, lanes = x.shape
    j = n // 2
    while j >= 1:
        m = n // (2 * j)
        x4 = x.reshape(rb, m, 2, j, lanes)
        x = jnp.concatenate([x4[:, :, 1:], x4[:, :, :1]],
                            axis=2).reshape(rb, n, lanes)
        j //= 2
    return x


def _sort_asc_refs(v_ref, i_ref):
    n = v_ref.shape[1]
    k = 2
    while k <= n:
        j = k // 2
        while j >= 1:
            v, ix = _stage(v_ref[...], i_ref[...], j, k, asc=True)
            v_ref[...] = v
            i_ref[...] = ix
            j //= 2
        k *= 2


def _cleanup_desc_refs(v_ref, i_ref):
    n = v_ref.shape[1]
    j = n // 2
    while j >= 1:
        v, ix = _stage(v_ref[...], i_ref[...], j, 2 * n, asc=False)
        v_ref[...] = v
        i_ref[...] = ix
        j //= 2


def _score_body(q_ref, k_ref, vals_ref, idx_ref, bv_ref, bi_ref,
                sv_ref, si_ref, *, num_keys, n_kb):
    c = pl.program_id(1)

    @pl.when(c == 0)
    def _init():
        sv_ref[...] = jnp.full(sv_ref.shape, _NEG, jnp.float32)
        si_ref[...] = jnp.full(si_ref.shape, _IMAX, jnp.int32)

    s = jax.lax.dot_general(
        q_ref[...], k_ref[...],
        dimension_numbers=(((1,), (1,)), ((), ())),
        preferred_element_type=jnp.float32,
    )
    rb = s.shape[0]
    s = s.reshape(rb, _DEPTH, 128)
    gidx = (c * _CB
            + jax.lax.broadcasted_iota(jnp.int32, s.shape, 1) * 128
            + jax.lax.broadcasted_iota(jnp.int32, s.shape, 2))
    bv_ref[...] = jnp.where(gidx < num_keys, s, _NEG)
    bi_ref[...] = gidx

    _sort_asc_refs(bv_ref, bi_ref)

    pad = 128 - _DEPTH
    bv = jnp.concatenate(
        [jnp.full((rb, pad, 128), _NEG, jnp.float32), bv_ref[...]], axis=1)
    bi = jnp.concatenate(
        [jnp.full((rb, pad, 128), _IMAX, jnp.int32), bi_ref[...]], axis=1)
    cb = _before(sv_ref[...], si_ref[...], bv, bi)
    sv_ref[...] = jnp.where(cb, sv_ref[...], bv)
    si_ref[...] = jnp.where(cb, si_ref[...], bi)
    _cleanup_desc_refs(sv_ref, si_ref)

    @pl.when(c == n_kb - 1)
    def _finalize():
        w = 64
        while w >= 1:
            av, ai = sv_ref[:, :, :w], si_ref[:, :, :w]
            bv = _rev(sv_ref[:, :, w:2 * w])
            bi = _rev(si_ref[:, :, w:2 * w])
            cb = _before(av, ai, bv, bi)
            sv_ref[:, :, :w] = jnp.where(cb, av, bv)
            si_ref[:, :, :w] = jnp.where(cb, ai, bi)
            n = sv_ref.shape[1]
            j = n // 2
            while j >= 1:
                v, ix = _stage(sv_ref[:, :, :w], si_ref[:, :, :w],
                               j, 2 * n, asc=False)
                sv_ref[:, :, :w] = v
                si_ref[:, :, :w] = ix
                j //= 2
            w //= 2
        vals_ref[...] = sv_ref[:, :, 0]
        idx_ref[...] = si_ref[:, :, 0]


def _run(q_index, keys, query_table, beam):
    batch = q_index.shape[0]
    num_keys, rank = keys.shape

    qt3 = query_table.reshape(query_table.shape[0], 1, rank)
    q = pl.pallas_call(
        _gather_body,
        grid_spec=pltpu.PrefetchScalarGridSpec(
            num_scalar_prefetch=1,
            grid=(batch,),
            in_specs=[pl.BlockSpec((1, 1, rank), lambda i, idx_ref: (idx_ref[i], 0, 0))],
            out_specs=pl.BlockSpec((1, 1, rank), lambda i, idx_ref: (i, 0, 0)),
        ),
        out_shape=jax.ShapeDtypeStruct((batch, 1, rank), jnp.float32),
    )(q_index, qt3).reshape(batch, rank)

    n_kb = pl.cdiv(num_keys, _CB)
    n_rb = batch // _RB
    vals, idx = pl.pallas_call(
        functools.partial(_score_body, num_keys=num_keys, n_kb=n_kb),
        grid=(n_rb, n_kb),
        in_specs=[
            pl.BlockSpec((_RB, rank), lambda r, c: (r, 0)),
            pl.BlockSpec((_CB, rank), lambda r, c: (c, 0)),
        ],
        out_specs=[
            pl.BlockSpec((_RB, _BEAM), lambda r, c: (r, 0)),
            pl.BlockSpec((_RB, _BEAM), lambda r, c: (r, 0)),
        ],
        out_shape=[
            jax.ShapeDtypeStruct((batch, _BEAM), jnp.float32),
            jax.ShapeDtypeStruct((batch, _BEAM), jnp.int32),
        ],
        scratch_shapes=[
            pltpu.VMEM((_RB, _DEPTH, 128), jnp.float32),
            pltpu.VMEM((_RB, _DEPTH, 128), jnp.int32),
            pltpu.VMEM((_RB, 128, 128), jnp.float32),
            pltpu.VMEM((_RB, 128, 128), jnp.int32),
        ],
    )(q, keys)

    return vals, idx + (beam - _BEAM)


def kernel(q_index, keys, query_table, beam):
    return _run(q_index, keys, query_table, beam)
